# baseline (speedup 1.0000x reference)
import functools

import jax
import jax.numpy as jnp
from jax import lax
from jax.experimental import pallas as pl
from jax.experimental.pallas import tpu as pltpu
from jax.experimental.pallas import tpu_sc as plsc

_NUM_E = 8192
_D = 64
_BC = 2048
_NCT = _NUM_E // _BC

_NW = 32
_N_POINTS = 16384
_BPW = _N_POINTS // _NW
_CH = 128
_NCH = _BPW // _CH


def _tc_argmin_body(z_ref, emb_ref, idx_ref, loss_ref,
                    mm_ref, esq_ref, min8_ref, chk8_ref):
    b = pl.program_id(0)
    c = pl.program_id(1)
    nb = pl.num_programs(0)
    HW = z_ref.shape[2]

    z = z_ref[0]
    emb = emb_ref[...]
    col_sq = jnp.sum(z * z, axis=0, keepdims=True)
    esq_ref[...] = jnp.sum(emb * emb, axis=1, keepdims=True)
    mm_ref[...] = lax.dot_general(
        emb * (-2.0), z, (((1,), (0,)), ((), ())),
        preferred_element_type=jnp.float32,
    )

    colsq8 = jnp.broadcast_to(col_sq, (8, HW))
    rmin0 = jnp.where(c == 0, jnp.full((8, HW), jnp.inf, jnp.float32),
                      min8_ref[...])
    rch0 = jnp.where(c == 0, jnp.zeros((8, HW), jnp.float32), chk8_ref[...])

    def chunk_body(j, carry):
        rmin, rch = carry
        s = pl.multiple_of(j * 8, 8)
        mmj = mm_ref[pl.ds(s, 8), :]
        esj = jnp.broadcast_to(esq_ref[pl.ds(s, 8), :], (8, HW))
        d = (colsq8 + mmj) + esj
        better = d < rmin
        chf = (c * (_BC // 8) + j).astype(jnp.float32)
        rch = jnp.where(better, jnp.full((8, HW), chf), rch)
        rmin = jnp.where(better, d, rmin)
        return rmin, rch

    rmin, rch = lax.fori_loop(0, _BC // 8, chunk_body, (rmin0, rch0),
                              unroll=4)
    min8_ref[...] = rmin
    chk8_ref[...] = rch

    @pl.when(c == _NCT - 1)
    def _finalize():
        tile_min = jnp.min(rmin, axis=0, keepdims=True)
        sub = lax.broadcasted_iota(jnp.int32, (8, HW), 0).astype(jnp.float32)
        rows = rch * 8.0 + sub
        cand = jnp.where(rmin == tile_min, rows, float(8 * _NUM_E))
        idx_ref[0] = jnp.min(cand, axis=0, keepdims=True).astype(jnp.int32)

        @pl.when(b == 0)
        def _zero():
            loss_ref[...] = jnp.zeros_like(loss_ref)

        loss_ref[...] = loss_ref[...] + jnp.sum(tile_min).reshape(1, 1)

        @pl.when(b == nb - 1)
        def _mean():
            loss_ref[...] = loss_ref[...] / float(_N_POINTS * _D)


def _tc_argmin(z3, emb):
    B, D, HW = z3.shape
    return pl.pallas_call(
        _tc_argmin_body,
        grid=(B, _NCT),
        in_specs=[
            pl.BlockSpec((1, D, HW), lambda b, c: (b, 0, 0)),
            pl.BlockSpec((_BC, D), lambda b, c: (c, 0)),
        ],
        out_specs=[
            pl.BlockSpec((1, 1, HW), lambda b, c: (b, 0, 0)),
            pl.BlockSpec((1, 1), lambda b, c: (0, 0)),
        ],
        out_shape=[
            jax.ShapeDtypeStruct((B, 1, HW), jnp.int32),
            jax.ShapeDtypeStruct((1, 1), jnp.float32),
        ],
        scratch_shapes=[
            pltpu.VMEM((_BC, HW), jnp.float32),
            pltpu.VMEM((_BC, 1), jnp.float32),
            pltpu.VMEM((8, HW), jnp.float32),
            pltpu.VMEM((8, HW), jnp.float32),
        ],
    )(z3, emb)


@functools.lru_cache(maxsize=None)
def _sc_gather_fn():
    def body(emb_hbm, idx_hbm, out_hbm, idx_v, rows_v, sem):
        wid = lax.axis_index("s") * 2 + lax.axis_index("c")
        pltpu.sync_copy(idx_hbm.at[wid], idx_v)
        copies = [
            pltpu.async_copy(emb_hbm.at[idx_v.at[j]],
                             rows_v.at[pl.ds(j * _CH, _CH)], sem)
            for j in range(_NCH)
        ]
        for cp in copies:
            cp.wait()
        pltpu.sync_copy(rows_v, out_hbm.at[wid])

    return pl.kernel(
        body,
        mesh=plsc.VectorSubcoreMesh(core_axis_name="c", subcore_axis_name="s"),
        out_type=jax.ShapeDtypeStruct((_NW, _BPW, _D), jnp.float32),
        scratch_types=[
            pltpu.VMEM((_NCH, _CH), jnp.int32),
            pltpu.VMEM((_BPW, _D), jnp.float32),
            pltpu.SemaphoreType.DMA,
        ],
        compiler_params=pltpu.CompilerParams(use_tc_tiling_on_sc=False),
    )


def kernel(z_e, embedding):
    B, D, H, W = z_e.shape
    HW = H * W
    z3 = z_e.reshape(B, D, HW)
    idx3, loss11 = _tc_argmin(z3, embedding)

    idx_flat = idx3.reshape(_NW, _NCH, _CH)
    zq_rows = _sc_gather_fn()(embedding, idx_flat)

    z_q = zq_rows.reshape(B, HW, D).transpose(0, 2, 1).reshape(B, D, H, W)
    return (z_q, loss11[0, 0], idx3.reshape(B, H, W))


# speedup vs baseline: 2.1401x; 2.1401x over previous
import functools

import jax
import jax.numpy as jnp
from jax import lax
from jax.experimental import pallas as pl
from jax.experimental.pallas import tpu as pltpu
from jax.experimental.pallas import tpu_sc as plsc

_NUM_E = 8192
_D = 64
_BC = 1024
_NCT = _NUM_E // _BC

_NW = 32
_N_POINTS = 16384
_BPW = _N_POINTS // _NW
_CH = 128
_NCH = _BPW // _CH


def _tc_argmin_body(z_ref, emb_ref, ids_ref, idx_ref, loss_ref, min_ref, arg_ref):
    b = pl.program_id(0)
    c = pl.program_id(1)
    nb = pl.num_programs(0)

    z = z_ref[0]
    emb = emb_ref[...]
    col_sq = jnp.sum(z * z, axis=0, keepdims=True)
    emb_sq = jnp.sum(emb * emb, axis=1, keepdims=True)
    mm2 = lax.dot_general(
        emb * (-2.0), z, (((1,), (0,)), ((), ())),
        preferred_element_type=jnp.float32,
    )
    dist = (col_sq + mm2) + emb_sq

    tile_min = jnp.min(dist, axis=0, keepdims=True)
    ids = jnp.broadcast_to(ids_ref[...], dist.shape)
    cand = jnp.where(dist == tile_min, ids, float(_NUM_E))
    tile_arg = jnp.min(cand, axis=0, keepdims=True) + float(_BC) * c

    @pl.when(c == 0)
    def _init():
        min_ref[...] = tile_min
        arg_ref[...] = tile_arg

    @pl.when(c > 0)
    def _merge():
        better = tile_min < min_ref[...]
        arg_ref[...] = jnp.where(better, tile_arg, arg_ref[...])
        min_ref[...] = jnp.where(better, tile_min, min_ref[...])

    @pl.when(c == _NCT - 1)
    def _finalize():
        idx_ref[0] = arg_ref[...].astype(jnp.int32)

        @pl.when(b == 0)
        def _zero():
            loss_ref[...] = jnp.zeros_like(loss_ref)

        loss_ref[...] = loss_ref[...] + jnp.sum(min_ref[...]).reshape(1, 1)

        @pl.when(b == nb - 1)
        def _mean():
            loss_ref[...] = loss_ref[...] / float(_N_POINTS * _D)


def _tc_argmin(z3, emb):
    B, D, HW = z3.shape
    ids_col = jnp.arange(_BC, dtype=jnp.float32).reshape(_BC, 1)
    return pl.pallas_call(
        _tc_argmin_body,
        grid=(B, _NCT),
        in_specs=[
            pl.BlockSpec((1, D, HW), lambda b, c: (b, 0, 0)),
            pl.BlockSpec((_BC, D), lambda b, c: (c, 0)),
            pl.BlockSpec((_BC, 1), lambda b, c: (0, 0)),
        ],
        out_specs=[
            pl.BlockSpec((1, 1, HW), lambda b, c: (b, 0, 0)),
            pl.BlockSpec((1, 1), lambda b, c: (0, 0)),
        ],
        out_shape=[
            jax.ShapeDtypeStruct((B, 1, HW), jnp.int32),
            jax.ShapeDtypeStruct((1, 1), jnp.float32),
        ],
        scratch_shapes=[
            pltpu.VMEM((1, HW), jnp.float32),
            pltpu.VMEM((1, HW), jnp.float32),
        ],
    )(z3, emb, ids_col)


@functools.lru_cache(maxsize=None)
def _sc_gather_fn():
    def body(emb_hbm, idx_hbm, out_hbm, idx_v, rows_v, sem):
        wid = lax.axis_index("s") * 2 + lax.axis_index("c")
        pltpu.sync_copy(idx_hbm.at[wid], idx_v)
        copies = [
            pltpu.async_copy(emb_hbm.at[idx_v.at[j]],
                             rows_v.at[pl.ds(j * _CH, _CH)], sem)
            for j in range(_NCH)
        ]
        for cp in copies:
            cp.wait()
        pltpu.sync_copy(rows_v, out_hbm.at[wid])

    return pl.kernel(
        body,
        mesh=plsc.VectorSubcoreMesh(core_axis_name="c", subcore_axis_name="s"),
        out_type=jax.ShapeDtypeStruct((_NW, _BPW, _D), jnp.float32),
        scratch_types=[
            pltpu.VMEM((_NCH, _CH), jnp.int32),
            pltpu.VMEM((_BPW, _D), jnp.float32),
            pltpu.SemaphoreType.DMA,
        ],
        compiler_params=pltpu.CompilerParams(use_tc_tiling_on_sc=False),
    )


def kernel(z_e, embedding):
    B, D, H, W = z_e.shape
    HW = H * W
    z3 = z_e.reshape(B, D, HW)
    idx3, loss11 = _tc_argmin(z3, embedding)

    idx_flat = idx3.reshape(_NW, _NCH, _CH)
    zq_rows = _sc_gather_fn()(embedding, idx_flat)

    z_q = zq_rows.reshape(B, HW, D).transpose(0, 2, 1).reshape(B, D, H, W)
    return (z_q, loss11[0, 0], idx3.reshape(B, H, W))


# speedup vs baseline: 2.3290x; 1.0883x over previous
import functools

import jax
import jax.numpy as jnp
from jax import lax
from jax.experimental import pallas as pl
from jax.experimental.pallas import tpu as pltpu
from jax.experimental.pallas import tpu_sc as plsc

_NUM_E = 8192
_D = 64
_BC = 4096
_NCT = _NUM_E // _BC

_NW = 32
_N_POINTS = 16384
_BPW = _N_POINTS // _NW
_CH = 128
_NCH = _BPW // _CH


def _tc_argmin_body(z_ref, emb_ref, ids_ref, idx_ref, loss_ref, min_ref, arg_ref):
    b = pl.program_id(0)
    c = pl.program_id(1)
    nb = pl.num_programs(0)

    z = z_ref[0]
    emb = emb_ref[...]
    col_sq = jnp.sum(z * z, axis=0, keepdims=True)
    emb_sq = jnp.sum(emb * emb, axis=1, keepdims=True)
    mm2 = lax.dot_general(
        emb * (-2.0), z, (((1,), (0,)), ((), ())),
        preferred_element_type=jnp.float32,
    )
    dist = (col_sq + mm2) + emb_sq

    tile_min = jnp.min(dist, axis=0, keepdims=True)
    ids = jnp.broadcast_to(ids_ref[...], dist.shape)
    cand = jnp.where(dist == tile_min, ids, float(_NUM_E))
    tile_arg = jnp.min(cand, axis=0, keepdims=True) + float(_BC) * c

    @pl.when(c == 0)
    def _init():
        min_ref[...] = tile_min
        arg_ref[...] = tile_arg

    @pl.when(c > 0)
    def _merge():
        better = tile_min < min_ref[...]
        arg_ref[...] = jnp.where(better, tile_arg, arg_ref[...])
        min_ref[...] = jnp.where(better, tile_min, min_ref[...])

    @pl.when(c == _NCT - 1)
    def _finalize():
        idx_ref[0] = arg_ref[...].astype(jnp.int32)

        @pl.when(b == 0)
        def _zero():
            loss_ref[...] = jnp.zeros_like(loss_ref)

        loss_ref[...] = loss_ref[...] + jnp.sum(min_ref[...]).reshape(1, 1)

        @pl.when(b == nb - 1)
        def _mean():
            loss_ref[...] = loss_ref[...] / float(_N_POINTS * _D)


def _tc_argmin(z3, emb):
    B, D, HW = z3.shape
    ids_col = jnp.arange(_BC, dtype=jnp.float32).reshape(_BC, 1)
    return pl.pallas_call(
        _tc_argmin_body,
        grid=(B, _NCT),
        in_specs=[
            pl.BlockSpec((1, D, HW), lambda b, c: (b, 0, 0)),
            pl.BlockSpec((_BC, D), lambda b, c: (c, 0)),
            pl.BlockSpec((_BC, 1), lambda b, c: (0, 0)),
        ],
        out_specs=[
            pl.BlockSpec((1, 1, HW), lambda b, c: (b, 0, 0)),
            pl.BlockSpec((1, 1), lambda b, c: (0, 0)),
        ],
        out_shape=[
            jax.ShapeDtypeStruct((B, 1, HW), jnp.int32),
            jax.ShapeDtypeStruct((1, 1), jnp.float32),
        ],
        scratch_shapes=[
            pltpu.VMEM((1, HW), jnp.float32),
            pltpu.VMEM((1, HW), jnp.float32),
        ],
    )(z3, emb, ids_col)


@functools.lru_cache(maxsize=None)
def _sc_gather_fn():
    def body(emb_hbm, idx_hbm, out_hbm, idx_v, rows_v, sem):
        wid = lax.axis_index("s") * 2 + lax.axis_index("c")
        pltpu.sync_copy(idx_hbm.at[wid], idx_v)
        copies = [
            pltpu.async_copy(emb_hbm.at[idx_v.at[j]],
                             rows_v.at[pl.ds(j * _CH, _CH)], sem)
            for j in range(_NCH)
        ]
        for cp in copies:
            cp.wait()
        pltpu.sync_copy(rows_v, out_hbm.at[wid])

    return pl.kernel(
        body,
        mesh=plsc.VectorSubcoreMesh(core_axis_name="c", subcore_axis_name="s"),
        out_type=jax.ShapeDtypeStruct((_NW, _BPW, _D), jnp.float32),
        scratch_types=[
            pltpu.VMEM((_NCH, _CH), jnp.int32),
            pltpu.VMEM((_BPW, _D), jnp.float32),
            pltpu.SemaphoreType.DMA,
        ],
        compiler_params=pltpu.CompilerParams(use_tc_tiling_on_sc=False),
    )


def kernel(z_e, embedding):
    B, D, H, W = z_e.shape
    HW = H * W
    z3 = z_e.reshape(B, D, HW)
    idx3, loss11 = _tc_argmin(z3, embedding)

    idx_flat = idx3.reshape(_NW, _NCH, _CH)
    zq_rows = _sc_gather_fn()(embedding, idx_flat)

    z_q = zq_rows.reshape(B, HW, D).transpose(0, 2, 1).reshape(B, D, H, W)
    return (z_q, loss11[0, 0], idx3.reshape(B, H, W))


# speedup vs baseline: 2.3524x; 1.0100x over previous
import functools

import jax
import jax.numpy as jnp
from jax import lax
from jax.experimental import pallas as pl
from jax.experimental.pallas import tpu as pltpu
from jax.experimental.pallas import tpu_sc as plsc

_NUM_E = 8192
_D = 64
_BC = 8192
_NCT = _NUM_E // _BC

_NW = 32
_N_POINTS = 16384
_BPW = _N_POINTS // _NW
_CH = 128
_NCH = _BPW // _CH


def _tc_argmin_body(z_ref, emb_ref, ids_ref, idx_ref, loss_ref, min_ref, arg_ref):
    b = pl.program_id(0)
    c = pl.program_id(1)
    nb = pl.num_programs(0)

    z = z_ref[0]
    emb = emb_ref[...]
    col_sq = jnp.sum(z * z, axis=0, keepdims=True)
    emb_sq = jnp.sum(emb * emb, axis=1, keepdims=True)
    mm2 = lax.dot_general(
        emb * (-2.0), z, (((1,), (0,)), ((), ())),
        preferred_element_type=jnp.float32,
    )
    dist = (col_sq + mm2) + emb_sq

    tile_min = jnp.min(dist, axis=0, keepdims=True)
    ids = jnp.broadcast_to(ids_ref[...], dist.shape)
    cand = jnp.where(dist == tile_min, ids, float(_NUM_E))
    tile_arg = jnp.min(cand, axis=0, keepdims=True) + float(_BC) * c

    @pl.when(c == 0)
    def _init():
        min_ref[...] = tile_min
        arg_ref[...] = tile_arg

    @pl.when(c > 0)
    def _merge():
        better = tile_min < min_ref[...]
        arg_ref[...] = jnp.where(better, tile_arg, arg_ref[...])
        min_ref[...] = jnp.where(better, tile_min, min_ref[...])

    @pl.when(c == _NCT - 1)
    def _finalize():
        idx_ref[0] = arg_ref[...].astype(jnp.int32)

        @pl.when(b == 0)
        def _zero():
            loss_ref[...] = jnp.zeros_like(loss_ref)

        loss_ref[...] = loss_ref[...] + jnp.sum(min_ref[...]).reshape(1, 1)

        @pl.when(b == nb - 1)
        def _mean():
            loss_ref[...] = loss_ref[...] / float(_N_POINTS * _D)


def _tc_argmin(z3, emb):
    B, D, HW = z3.shape
    ids_col = jnp.arange(_BC, dtype=jnp.float32).reshape(_BC, 1)
    return pl.pallas_call(
        _tc_argmin_body,
        grid=(B, _NCT),
        in_specs=[
            pl.BlockSpec((1, D, HW), lambda b, c: (b, 0, 0)),
            pl.BlockSpec((_BC, D), lambda b, c: (c, 0)),
            pl.BlockSpec((_BC, 1), lambda b, c: (0, 0)),
        ],
        out_specs=[
            pl.BlockSpec((1, 1, HW), lambda b, c: (b, 0, 0)),
            pl.BlockSpec((1, 1), lambda b, c: (0, 0)),
        ],
        out_shape=[
            jax.ShapeDtypeStruct((B, 1, HW), jnp.int32),
            jax.ShapeDtypeStruct((1, 1), jnp.float32),
        ],
        scratch_shapes=[
            pltpu.VMEM((1, HW), jnp.float32),
            pltpu.VMEM((1, HW), jnp.float32),
        ],
    )(z3, emb, ids_col)


@functools.lru_cache(maxsize=None)
def _sc_gather_fn():
    def body(emb_hbm, idx_hbm, out_hbm, idx_v, rows_v, sem):
        wid = lax.axis_index("s") * 2 + lax.axis_index("c")
        pltpu.sync_copy(idx_hbm.at[wid], idx_v)
        copies = [
            pltpu.async_copy(emb_hbm.at[idx_v.at[j]],
                             rows_v.at[pl.ds(j * _CH, _CH)], sem)
            for j in range(_NCH)
        ]
        for cp in copies:
            cp.wait()
        pltpu.sync_copy(rows_v, out_hbm.at[wid])

    return pl.kernel(
        body,
        mesh=plsc.VectorSubcoreMesh(core_axis_name="c", subcore_axis_name="s"),
        out_type=jax.ShapeDtypeStruct((_NW, _BPW, _D), jnp.float32),
        scratch_types=[
            pltpu.VMEM((_NCH, _CH), jnp.int32),
            pltpu.VMEM((_BPW, _D), jnp.float32),
            pltpu.SemaphoreType.DMA,
        ],
        compiler_params=pltpu.CompilerParams(use_tc_tiling_on_sc=False),
    )


def kernel(z_e, embedding):
    B, D, H, W = z_e.shape
    HW = H * W
    z3 = z_e.reshape(B, D, HW)
    idx3, loss11 = _tc_argmin(z3, embedding)

    idx_flat = idx3.reshape(_NW, _NCH, _CH)
    zq_rows = _sc_gather_fn()(embedding, idx_flat)

    z_q = zq_rows.reshape(B, HW, D).transpose(0, 2, 1).reshape(B, D, H, W)
    return (z_q, loss11[0, 0], idx3.reshape(B, H, W))


# speedup vs baseline: 2.3542x; 1.0008x over previous
import functools

import jax
import jax.numpy as jnp
from jax import lax
from jax.experimental import pallas as pl
from jax.experimental.pallas import tpu as pltpu
from jax.experimental.pallas import tpu_sc as plsc

_NUM_E = 8192
_D = 64
_BC = 8192
_NCT = _NUM_E // _BC

_NW = 32
_N_POINTS = 16384
_BPW = _N_POINTS // _NW
_CH = 128
_NCH = _BPW // _CH


def _tc_argmin_body(z_ref, emb_ref, ids_ref, idx_ref, loss_ref, min_ref, arg_ref):
    b = pl.program_id(0)
    c = pl.program_id(1)
    nb = pl.num_programs(0)

    z = z_ref[0]
    emb = emb_ref[...]
    col_sq = jnp.sum(z * z, axis=0, keepdims=True)
    emb_sq = jnp.sum(emb * emb, axis=1, keepdims=True)
    mm2 = lax.dot_general(
        emb * (-2.0), z, (((1,), (0,)), ((), ())),
        preferred_element_type=jnp.float32,
    )
    dist = (col_sq + mm2) + emb_sq

    tile_min = jnp.min(dist, axis=0, keepdims=True)
    cand = jnp.where(dist == tile_min, ids_ref[...], float(_NUM_E))
    tile_arg = jnp.min(cand, axis=0, keepdims=True) + float(_BC) * c

    @pl.when(c == 0)
    def _init():
        min_ref[...] = tile_min
        arg_ref[...] = tile_arg

    @pl.when(c > 0)
    def _merge():
        better = tile_min < min_ref[...]
        arg_ref[...] = jnp.where(better, tile_arg, arg_ref[...])
        min_ref[...] = jnp.where(better, tile_min, min_ref[...])

    @pl.when(c == _NCT - 1)
    def _finalize():
        idx_ref[0] = arg_ref[...].astype(jnp.int32)

        @pl.when(b == 0)
        def _zero():
            loss_ref[...] = jnp.zeros_like(loss_ref)

        loss_ref[...] = loss_ref[...] + jnp.sum(min_ref[...]).reshape(1, 1)

        @pl.when(b == nb - 1)
        def _mean():
            loss_ref[...] = loss_ref[...] / float(_N_POINTS * _D)


def _tc_argmin(z3, emb):
    B, D, HW = z3.shape
    ids_col = jnp.arange(_BC, dtype=jnp.float32).reshape(_BC, 1)
    return pl.pallas_call(
        _tc_argmin_body,
        grid=(B, _NCT),
        in_specs=[
            pl.BlockSpec((1, D, HW), lambda b, c: (b, 0, 0)),
            pl.BlockSpec((_BC, D), lambda b, c: (c, 0)),
            pl.BlockSpec((_BC, 1), lambda b, c: (0, 0)),
        ],
        out_specs=[
            pl.BlockSpec((1, 1, HW), lambda b, c: (b, 0, 0)),
            pl.BlockSpec((1, 1), lambda b, c: (0, 0)),
        ],
        out_shape=[
            jax.ShapeDtypeStruct((B, 1, HW), jnp.int32),
            jax.ShapeDtypeStruct((1, 1), jnp.float32),
        ],
        scratch_shapes=[
            pltpu.VMEM((1, HW), jnp.float32),
            pltpu.VMEM((1, HW), jnp.float32),
        ],
    )(z3, emb, ids_col)


@functools.lru_cache(maxsize=None)
def _sc_gather_fn():
    def body(emb_hbm, idx_hbm, out_hbm, idx_v, rows_v, sem):
        wid = lax.axis_index("s") * 2 + lax.axis_index("c")
        pltpu.sync_copy(idx_hbm.at[wid], idx_v)
        copies = [
            pltpu.async_copy(emb_hbm.at[idx_v.at[j]],
                             rows_v.at[pl.ds(j * _CH, _CH)], sem)
            for j in range(_NCH)
        ]
        for cp in copies:
            cp.wait()
        pltpu.sync_copy(rows_v, out_hbm.at[wid])

    return pl.kernel(
        body,
        mesh=plsc.VectorSubcoreMesh(core_axis_name="c", subcore_axis_name="s"),
        out_type=jax.ShapeDtypeStruct((_NW, _BPW, _D), jnp.float32),
        scratch_types=[
            pltpu.VMEM((_NCH, _CH), jnp.int32),
            pltpu.VMEM((_BPW, _D), jnp.float32),
            pltpu.SemaphoreType.DMA,
        ],
        compiler_params=pltpu.CompilerParams(use_tc_tiling_on_sc=False),
    )


def kernel(z_e, embedding):
    B, D, H, W = z_e.shape
    HW = H * W
    z3 = z_e.reshape(B, D, HW)
    idx3, loss11 = _tc_argmin(z3, embedding)

    idx_flat = idx3.reshape(_NW, _NCH, _CH)
    zq_rows = _sc_gather_fn()(embedding, idx_flat)

    z_q = zq_rows.reshape(B, HW, D).transpose(0, 2, 1).reshape(B, D, H, W)
    return (z_q, loss11[0, 0], idx3.reshape(B, H, W))
